# baseline (speedup 1.0000x reference)
import functools

import jax
import jax.numpy as jnp
from jax import lax
from jax.experimental import pallas as pl
from jax.experimental.pallas import tpu as pltpu
from jax.experimental.pallas import tpu_sc as plsc

_LANES = 16
_NWORKERS = 32
_ROWS_PER = 8
_L0DIM = 128
_HID = 64
_VPAD = 128


def _mlp_rows(nrows, embc_v, w1_v, b1_v, w2_v, b2s, tabw_v):
    nj = _HID // _LANES
    grp = 4
    assert nrows % grp == 0
    lane_iota = lax.iota(jnp.int32, _LANES)
    tabvec = jnp.zeros((_LANES,), jnp.float32)

    for g in range(nrows // grp):
        rows = [g * grp + rr for rr in range(grp)]

        def body(t, hs):
            hs = [list(hrow) for hrow in hs]
            evs = [embc_v[pl.ds(r * _L0DIM + t * _LANES, _LANES)]
                   for r in rows]
            for l in range(_LANES):
                koff = (t * _LANES + l) * _HID
                w1s = [w1_v[pl.ds(koff + j * _LANES, _LANES)]
                       for j in range(nj)]
                for rr in range(grp):
                    e = evs[rr][l]
                    for j in range(nj):
                        hs[rr][j] = hs[rr][j] + e * w1s[j]
            return tuple(tuple(hrow) for hrow in hs)

        hs0 = tuple(tuple(jnp.zeros((_LANES,), jnp.float32)
                          for _ in range(nj)) for _ in range(grp))
        hs = lax.fori_loop(0, _L0DIM // _LANES, body, hs0)

        for rr in range(grp):
            acc = jnp.zeros((_LANES,), jnp.float32)
            for j in range(nj):
                h = hs[rr][j] + b1_v[pl.ds(j * _LANES, _LANES)]
                sig = 1.0 / (1.0 + jnp.exp(-h))
                acc = acc + (h * sig) * w2_v[pl.ds(j * _LANES, _LANES)]
            srow = jnp.sum(acc) + b2s
            rmask = (lane_iota == rows[rr]).astype(jnp.float32)
            tabvec = tabvec + srow * rmask
    tabw_v[...] = tabvec


def _make_fused(n, vocab):
    chunk = -(-n // _NWORKERS)
    chunk = -(-chunk // (4 * _LANES)) * (4 * _LANES)
    tail = n - (_NWORKERS - 1) * chunk
    assert 0 < tail <= chunk and chunk % (4 * _LANES) == 0
    assert tail % (2 * _LANES) == 0
    nfull, rem = divmod(vocab, _ROWS_PER)
    assert (nfull + (rem > 0)) <= 14 and vocab <= _VPAD

    mesh = plsc.VectorSubcoreMesh(core_axis_name="c", subcore_axis_name="s")

    @functools.partial(
        pl.kernel,
        out_type=jax.ShapeDtypeStruct((n,), jnp.float32),
        mesh=mesh,
        scratch_types=[
            pltpu.VMEM((chunk,), jnp.int32),
            pltpu.VMEM((chunk,), jnp.float32),
            pltpu.VMEM((_VPAD,), jnp.float32),
            pltpu.VMEM((_ROWS_PER * _L0DIM,), jnp.float32),
            pltpu.VMEM((_L0DIM * _HID,), jnp.float32),
            pltpu.VMEM((_HID,), jnp.float32),
            pltpu.VMEM((_HID,), jnp.float32),
            pltpu.VMEM((_LANES,), jnp.float32),
            pltpu.VMEM((_LANES,), jnp.float32),
            pltpu.VMEM_SHARED((_VPAD,), jnp.float32),
            pltpu.SemaphoreType.DMA,
        ],
        compiler_params=pltpu.CompilerParams(needs_layout_passes=False),
    )
    def fused(z_hbm, emb_hbm, w1_hbm, b1_hbm, w2_hbm, b2_hbm, out_hbm,
              idx_v, val_v, tab_v, embc_v, w1_v, b1_v, w2_v, b2_v, tabw_v,
              tab_sh, sem_z):
        s = lax.axis_index("s")
        wid = s * 2 + lax.axis_index("c")
        base = wid * chunk
        is_main = wid < _NWORKERS - 1

        @pl.when(is_main)
        def _():
            pltpu.async_copy(z_hbm.at[pl.ds(base, chunk)], idx_v, sem_z)

        @pl.when(jnp.logical_not(is_main))
        def _():
            pltpu.async_copy(z_hbm.at[pl.ds(base, tail)],
                             idx_v.at[pl.ds(0, tail)], sem_z)

        def build(nrows):
            pltpu.sync_copy(
                emb_hbm.at[pl.ds(s * (_ROWS_PER * _L0DIM), nrows * _L0DIM)],
                embc_v.at[pl.ds(0, nrows * _L0DIM)])
            pltpu.sync_copy(w1_hbm, w1_v)
            pltpu.sync_copy(b1_hbm, b1_v)
            pltpu.sync_copy(w2_hbm, w2_v)
            pltpu.sync_copy(b2_hbm, b2_v.at[pl.ds(0, 1)])
            b2s = b2_v[...][0]
            _mlp_rows(nrows, embc_v, w1_v, b1_v, w2_v, b2s, tabw_v)
            pltpu.sync_copy(tabw_v.at[pl.ds(0, _ROWS_PER)],
                            tab_sh.at[pl.ds(s * _ROWS_PER, _ROWS_PER)])

        @pl.when(s < nfull)
        def _():
            build(_ROWS_PER)

        if rem:
            @pl.when(s == nfull)
            def _():
                build(rem)

        plsc.subcore_barrier()
        pltpu.sync_copy(tab_sh, tab_v)

        def gather(count, unroll):
            def body(i, carry):
                o = i * (_LANES * unroll)
                for u in range(unroll):
                    q = o + u * _LANES
                    idx = idx_v[pl.ds(q, _LANES)]
                    val_v[pl.ds(q, _LANES)] = plsc.load_gather(tab_v, [idx])
                return carry
            lax.fori_loop(0, count // unroll, body, 0)

        @pl.when(is_main)
        def _():
            pltpu.make_async_copy(
                z_hbm.at[pl.ds(base, chunk)], idx_v, sem_z).wait()
            gather(chunk // _LANES, 4)
            pltpu.sync_copy(val_v, out_hbm.at[pl.ds(base, chunk)])

        @pl.when(jnp.logical_not(is_main))
        def _():
            pltpu.make_async_copy(
                z_hbm.at[pl.ds(base, tail)],
                idx_v.at[pl.ds(0, tail)], sem_z).wait()
            gather(tail // _LANES, 2)
            pltpu.sync_copy(val_v.at[pl.ds(0, tail)],
                            out_hbm.at[pl.ds(base, tail)])

    return fused


def kernel(z, batch, pos, emb_table, W1, b1, W2, b2):
    n = z.shape[0]
    vocab = emb_table.shape[0]
    outp = _make_fused(n, vocab)(
        z.astype(jnp.int32),
        emb_table.reshape(-1),
        W1.reshape(-1),
        b1,
        W2.reshape(-1),
        b2,
    )
    return outp.reshape(n, 1)


# speedup vs baseline: 1.4911x; 1.4911x over previous
import functools

import jax
import jax.numpy as jnp
from jax import lax
from jax.experimental import pallas as pl
from jax.experimental.pallas import tpu as pltpu
from jax.experimental.pallas import tpu_sc as plsc

_LANES = 16
_NWORKERS = 32


def _mlp_body(emb_ref, w1_ref, b1_ref, w2t_ref, b2_ref, out_ref):
    h = jnp.dot(emb_ref[...], w1_ref[...], preferred_element_type=jnp.float32)
    h = h + b1_ref[...]
    h = h * jax.nn.sigmoid(h)
    tab = lax.dot_general(w2t_ref[...], h, (((1,), (1,)), ((), ())),
                          preferred_element_type=jnp.float32)
    out_ref[...] = tab + b2_ref[0, 0]


def _vocab_mlp(emb_table, W1, b1, W2, b2):
    vocab = emb_table.shape[0]
    tab2 = pl.pallas_call(
        _mlp_body,
        out_shape=jax.ShapeDtypeStruct((1, vocab), jnp.float32),
    )(emb_table, W1, b1.reshape(1, -1), W2.reshape(1, -1), b2.reshape(1, 1))
    return tab2.reshape(vocab)


def _gather_loop(tab_v, idx_v, val_v, count, unroll):
    def body(i, carry):
        s = i * (_LANES * unroll)
        for u in range(unroll):
            o = s + u * _LANES
            idx = idx_v[pl.ds(o, _LANES)]
            val_v[pl.ds(o, _LANES)] = plsc.load_gather(tab_v, [idx])
        return carry

    lax.fori_loop(0, count // unroll, body, 0)


def _make_sc_gather(n, vocab):
    chunk = -(-n // _NWORKERS)
    chunk = -(-chunk // (4 * _LANES)) * (4 * _LANES)
    tail = n - (_NWORKERS - 1) * chunk
    assert 0 < tail <= chunk and chunk % (4 * _LANES) == 0
    assert tail % (2 * _LANES) == 0

    mesh = plsc.VectorSubcoreMesh(core_axis_name="c", subcore_axis_name="s")

    @functools.partial(
        pl.kernel,
        out_type=jax.ShapeDtypeStruct((n,), jnp.float32),
        mesh=mesh,
        scratch_types=[
            pltpu.VMEM((chunk,), jnp.int32),
            pltpu.VMEM((chunk,), jnp.float32),
            pltpu.VMEM((vocab,), jnp.float32),
            pltpu.SemaphoreType.DMA,
        ],
        compiler_params=pltpu.CompilerParams(needs_layout_passes=False),
    )
    def sc_gather(z_hbm, tab_hbm, out_hbm, idx_v, val_v, tab_v, sem):
        wid = lax.axis_index("s") * 2 + lax.axis_index("c")
        base = wid * chunk
        is_main = wid < _NWORKERS - 1

        @pl.when(is_main)
        def _():
            cp = pltpu.async_copy(z_hbm.at[pl.ds(base, chunk)], idx_v, sem)
            pltpu.sync_copy(tab_hbm, tab_v)
            cp.wait()
            _gather_loop(tab_v, idx_v, val_v, chunk // _LANES, 4)
            pltpu.sync_copy(val_v, out_hbm.at[pl.ds(base, chunk)])

        @pl.when(jnp.logical_not(is_main))
        def _():
            idx_t = idx_v.at[pl.ds(0, tail)]
            val_t = val_v.at[pl.ds(0, tail)]
            cp = pltpu.async_copy(z_hbm.at[pl.ds(base, tail)], idx_t, sem)
            pltpu.sync_copy(tab_hbm, tab_v)
            cp.wait()
            _gather_loop(tab_v, idx_v, val_v, tail // _LANES, 2)
            pltpu.sync_copy(val_t, out_hbm.at[pl.ds(base, tail)])

    return sc_gather


def kernel(z, batch, pos, emb_table, W1, b1, W2, b2):
    n = z.shape[0]
    vocab = emb_table.shape[0]
    tab = _vocab_mlp(emb_table, W1, b1, W2, b2)
    outp = _make_sc_gather(n, vocab)(z.astype(jnp.int32), tab)
    return outp.reshape(n, 1)


# speedup vs baseline: 1.4952x; 1.0028x over previous
import functools

import jax
import jax.numpy as jnp
from jax import lax
from jax.experimental import pallas as pl
from jax.experimental.pallas import tpu as pltpu
from jax.experimental.pallas import tpu_sc as plsc

_LANES = 16
_NWORKERS = 32


def _mlp_body(emb_ref, w1_ref, b1_ref, w2t_ref, b2_ref, out_ref):
    h = jnp.dot(emb_ref[...], w1_ref[...], preferred_element_type=jnp.float32)
    h = h + b1_ref[...]
    h = h * jax.nn.sigmoid(h)
    tab = lax.dot_general(w2t_ref[...], h, (((1,), (1,)), ((), ())),
                          preferred_element_type=jnp.float32)
    out_ref[...] = tab + b2_ref[0, 0]


def _vocab_mlp(emb_table, W1, b1, W2, b2):
    vocab = emb_table.shape[0]
    tab2 = pl.pallas_call(
        _mlp_body,
        out_shape=jax.ShapeDtypeStruct((1, vocab), jnp.float32),
    )(emb_table, W1, b1.reshape(1, -1), W2.reshape(1, -1), b2.reshape(1, 1))
    return tab2.reshape(vocab)


def _gather_loop(tab_v, idx_v, val_v, count, unroll):
    def body(i, carry):
        s = i * (_LANES * unroll)
        for u in range(unroll):
            o = s + u * _LANES
            idx = idx_v[pl.ds(o, _LANES)]
            val_v[pl.ds(o, _LANES)] = plsc.load_gather(tab_v, [idx])
        return carry

    lax.fori_loop(0, count // unroll, body, 0)


def _make_sc_gather(n, vocab):
    chunk = -(-n // _NWORKERS)
    chunk = -(-chunk // (4 * _LANES)) * (4 * _LANES)
    tail = n - (_NWORKERS - 1) * chunk
    assert 0 < tail <= chunk and chunk % (4 * _LANES) == 0
    assert tail % (2 * _LANES) == 0

    mesh = plsc.VectorSubcoreMesh(core_axis_name="c", subcore_axis_name="s")

    @functools.partial(
        pl.kernel,
        out_type=jax.ShapeDtypeStruct((n,), jnp.float32),
        mesh=mesh,
        scratch_types=[
            pltpu.VMEM((chunk,), jnp.int32),
            pltpu.VMEM((chunk,), jnp.float32),
            pltpu.VMEM((vocab,), jnp.float32),
            pltpu.SemaphoreType.DMA,
        ],
        compiler_params=pltpu.CompilerParams(
            needs_layout_passes=False,
            disable_bounds_checks=True,
            disable_semaphore_checks=True,
            skip_device_barrier=True,
        ),
    )
    def sc_gather(z_hbm, tab_hbm, out_hbm, idx_v, val_v, tab_v, sem):
        wid = lax.axis_index("s") * 2 + lax.axis_index("c")
        base = wid * chunk
        is_main = wid < _NWORKERS - 1

        @pl.when(is_main)
        def _():
            cp = pltpu.async_copy(z_hbm.at[pl.ds(base, chunk)], idx_v, sem)
            pltpu.sync_copy(tab_hbm, tab_v)
            cp.wait()
            _gather_loop(tab_v, idx_v, val_v, chunk // _LANES, 4)
            pltpu.sync_copy(val_v, out_hbm.at[pl.ds(base, chunk)])

        @pl.when(jnp.logical_not(is_main))
        def _():
            idx_t = idx_v.at[pl.ds(0, tail)]
            val_t = val_v.at[pl.ds(0, tail)]
            cp = pltpu.async_copy(z_hbm.at[pl.ds(base, tail)], idx_t, sem)
            pltpu.sync_copy(tab_hbm, tab_v)
            cp.wait()
            _gather_loop(tab_v, idx_v, val_v, tail // _LANES, 2)
            pltpu.sync_copy(val_t, out_hbm.at[pl.ds(base, tail)])

    return sc_gather


def kernel(z, batch, pos, emb_table, W1, b1, W2, b2):
    n = z.shape[0]
    vocab = emb_table.shape[0]
    tab = _vocab_mlp(emb_table, W1, b1, W2, b2)
    outp = _make_sc_gather(n, vocab)(z.astype(jnp.int32), tab)
    return outp.reshape(n, 1)


# speedup vs baseline: 1.5513x; 1.0375x over previous
import functools

import jax
import jax.numpy as jnp
from jax import lax
from jax.experimental import pallas as pl
from jax.experimental.pallas import tpu as pltpu
from jax.experimental.pallas import tpu_sc as plsc

_LANES = 16
_NWORKERS = 16


def _mlp_body(emb_ref, w1_ref, b1_ref, w2t_ref, b2_ref, out_ref):
    h = jnp.dot(emb_ref[...], w1_ref[...], preferred_element_type=jnp.float32)
    h = h + b1_ref[...]
    h = h * jax.nn.sigmoid(h)
    tab = lax.dot_general(w2t_ref[...], h, (((1,), (1,)), ((), ())),
                          preferred_element_type=jnp.float32)
    out_ref[...] = tab + b2_ref[0, 0]


def _vocab_mlp(emb_table, W1, b1, W2, b2):
    vocab = emb_table.shape[0]
    tab2 = pl.pallas_call(
        _mlp_body,
        out_shape=jax.ShapeDtypeStruct((1, vocab), jnp.float32),
    )(emb_table, W1, b1.reshape(1, -1), W2.reshape(1, -1), b2.reshape(1, 1))
    return tab2.reshape(vocab)


def _gather_loop(tab_v, idx_v, val_v, count, unroll):
    def body(i, carry):
        s = i * (_LANES * unroll)
        for u in range(unroll):
            o = s + u * _LANES
            idx = idx_v[pl.ds(o, _LANES)]
            val_v[pl.ds(o, _LANES)] = plsc.load_gather(tab_v, [idx])
        return carry

    lax.fori_loop(0, count // unroll, body, 0)


def _make_sc_gather(n, vocab):
    chunk = -(-n // _NWORKERS)
    chunk = -(-chunk // (4 * _LANES)) * (4 * _LANES)
    tail = n - (_NWORKERS - 1) * chunk
    assert 0 < tail <= chunk and chunk % (4 * _LANES) == 0
    assert tail % (2 * _LANES) == 0

    mesh = plsc.VectorSubcoreMesh(core_axis_name="c", subcore_axis_name="s", num_cores=1)

    @functools.partial(
        pl.kernel,
        out_type=jax.ShapeDtypeStruct((n,), jnp.float32),
        mesh=mesh,
        scratch_types=[
            pltpu.VMEM((chunk,), jnp.int32),
            pltpu.VMEM((chunk,), jnp.float32),
            pltpu.VMEM((vocab,), jnp.float32),
            pltpu.SemaphoreType.DMA,
        ],
        compiler_params=pltpu.CompilerParams(needs_layout_passes=False),
    )
    def sc_gather(z_hbm, tab_hbm, out_hbm, idx_v, val_v, tab_v, sem):
        wid = lax.axis_index("s")
        base = wid * chunk
        is_main = wid < _NWORKERS - 1

        @pl.when(is_main)
        def _():
            cp = pltpu.async_copy(z_hbm.at[pl.ds(base, chunk)], idx_v, sem)
            pltpu.sync_copy(tab_hbm, tab_v)
            cp.wait()
            _gather_loop(tab_v, idx_v, val_v, chunk // _LANES, 4)
            pltpu.sync_copy(val_v, out_hbm.at[pl.ds(base, chunk)])

        @pl.when(jnp.logical_not(is_main))
        def _():
            idx_t = idx_v.at[pl.ds(0, tail)]
            val_t = val_v.at[pl.ds(0, tail)]
            cp = pltpu.async_copy(z_hbm.at[pl.ds(base, tail)], idx_t, sem)
            pltpu.sync_copy(tab_hbm, tab_v)
            cp.wait()
            _gather_loop(tab_v, idx_v, val_v, tail // _LANES, 2)
            pltpu.sync_copy(val_t, out_hbm.at[pl.ds(base, tail)])

    return sc_gather


def kernel(z, batch, pos, emb_table, W1, b1, W2, b2):
    n = z.shape[0]
    vocab = emb_table.shape[0]
    tab = _vocab_mlp(emb_table, W1, b1, W2, b2)
    outp = _make_sc_gather(n, vocab)(z.astype(jnp.int32), tab)
    return outp.reshape(n, 1)


# speedup vs baseline: 1.5608x; 1.0061x over previous
import functools

import jax
import jax.numpy as jnp
from jax import lax
from jax.experimental import pallas as pl
from jax.experimental.pallas import tpu as pltpu
from jax.experimental.pallas import tpu_sc as plsc

_LANES = 16
_NWORKERS = 16


def _mlp_body(emb_ref, w1_ref, b1_ref, w2t_ref, b2_ref, out_ref):
    h = jnp.dot(emb_ref[...], w1_ref[...], preferred_element_type=jnp.float32)
    h = h + b1_ref[...]
    h = h * jax.nn.sigmoid(h)
    tab = lax.dot_general(w2t_ref[...], h, (((1,), (1,)), ((), ())),
                          preferred_element_type=jnp.float32)
    out_ref[...] = tab + b2_ref[0, 0]


def _vocab_mlp(emb_table, W1, b1, W2, b2):
    vocab = emb_table.shape[0]
    tab2 = pl.pallas_call(
        _mlp_body,
        out_shape=jax.ShapeDtypeStruct((1, vocab), jnp.float32),
    )(emb_table, W1, b1.reshape(1, -1), W2.reshape(1, -1), b2.reshape(1, 1))
    return tab2.reshape(vocab)


def _gather_loop(tab_v, idx_v, val_v, count, unroll):
    def body(i, carry):
        s = i * (_LANES * unroll)
        for u in range(unroll):
            o = s + u * _LANES
            idx = idx_v[pl.ds(o, _LANES)]
            val_v[pl.ds(o, _LANES)] = plsc.load_gather(tab_v, [idx])
        return carry

    lax.fori_loop(0, count // unroll, body, 0)


def _make_sc_gather(n, vocab):
    chunk = -(-n // _NWORKERS)
    chunk = -(-chunk // (4 * _LANES)) * (4 * _LANES)
    assert chunk <= n and chunk % (4 * _LANES) == 0
    assert (n - chunk) % _LANES == 0

    mesh = plsc.VectorSubcoreMesh(core_axis_name="c", subcore_axis_name="s", num_cores=1)

    @functools.partial(
        pl.kernel,
        out_type=jax.ShapeDtypeStruct((n,), jnp.float32),
        mesh=mesh,
        scratch_types=[
            pltpu.VMEM((chunk,), jnp.int32),
            pltpu.VMEM((chunk,), jnp.float32),
            pltpu.VMEM((vocab,), jnp.float32),
            pltpu.SemaphoreType.DMA,
        ],
        compiler_params=pltpu.CompilerParams(needs_layout_passes=False),
    )
    def sc_gather(z_hbm, tab_hbm, out_hbm, idx_v, val_v, tab_v, sem):
        wid = lax.axis_index("s")
        base = jnp.minimum(wid * chunk, n - chunk)
        cp = pltpu.async_copy(z_hbm.at[pl.ds(base, chunk)], idx_v, sem)
        pltpu.sync_copy(tab_hbm, tab_v)
        cp.wait()
        _gather_loop(tab_v, idx_v, val_v, chunk // _LANES, 4)
        pltpu.sync_copy(val_v, out_hbm.at[pl.ds(base, chunk)])

    return sc_gather


def kernel(z, batch, pos, emb_table, W1, b1, W2, b2):
    n = z.shape[0]
    vocab = emb_table.shape[0]
    tab = _vocab_mlp(emb_table, W1, b1, W2, b2)
    outp = _make_sc_gather(n, vocab)(z.astype(jnp.int32), tab)
    return outp.reshape(n, 1)
